# baseline (speedup 1.0000x reference)
import functools

import jax
import jax.numpy as jnp
import numpy as np
from jax.experimental import pallas as pl
from jax.experimental.pallas import tpu as pltpu

EPS = 1e-5
VMEM_LIMIT = 64 * 1024 * 1024


def _fused_kernel(canvas_ref, w1_ref, sc1_ref, sh1_ref, sel_ref,
                  w2_ref, sh2_ref, w3_ref, sc3_ref, sh3_ref,
                  wfc_ref, fcb_ref, out_ref,
                  col_ref, c2_ref, c3_ref, stack_ref, facc_ref):
    f32 = jnp.float32
    bf16 = jnp.bfloat16

    rl_in, rl_out = 256, 128
    cb1 = 2048
    w1 = w1_ref[...]
    sc1 = sc1_ref[...]
    sh1 = sh1_ref[...]
    sel = sel_ref[...]

    c2_ref[:, pl.ds(0, 2 * rl_out)] = jnp.zeros((16, 2 * rl_out), bf16)
    c2_ref[:, pl.ds(98 * rl_out, 2 * rl_out)] = jnp.zeros((16, 2 * rl_out), bf16)

    for g in range(24):
        slot = g % 2
        o = 2 * rl_in + g * cb1
        for dy in range(3):
            for dx in range(3):
                t = dy * 3 + dx
                col_ref[slot, pl.ds(t * 8, 8), :] = canvas_ref[
                    0, :, pl.ds(o + (dy - 1) * rl_in + (dx - 1), cb1)]
        y = jnp.dot(w1, col_ref[slot, pl.ds(0, 72), :],
                    preferred_element_type=f32)
        y = jnp.maximum(y * sc1 + sh1, 0.0)
        for p in range(4):
            a = y[:, (2 * p) * rl_in:(2 * p + 1) * rl_in]
            b = y[:, (2 * p + 1) * rl_in:(2 * p + 2) * rl_in]
            stack_ref[pl.ds(p * 16, 16), :] = jnp.maximum(a, b).astype(bf16)
        res = jnp.dot(stack_ref[...], sel, preferred_element_type=f32)
        pooled = jnp.maximum(res[:, :rl_out], res[:, rl_out:])
        for p in range(4):
            yq = 2 + 4 * g + p
            c2_ref[:, pl.ds(yq * rl_out, rl_out)] = pooled[
                p * 16:(p + 1) * 16, :].astype(bf16)

    cb = 2048
    w2 = w2_ref[...]
    sh2 = sh2_ref[...]
    lane = jax.lax.broadcasted_iota(jnp.int32, (1, cb), 1) % rl_out
    keep = jnp.logical_and(lane >= 1, lane <= 96).astype(f32)

    c3_ref[:, pl.ds(0, 2 * rl_out)] = jnp.zeros((24, 2 * rl_out), bf16)
    c3_ref[:, pl.ds(98 * rl_out, 2 * rl_out)] = jnp.zeros((24, 2 * rl_out), bf16)

    for j in range(6):
        slot = j % 2
        o = 2 * rl_out + j * cb
        for dy in range(3):
            for dx in range(3):
                t = dy * 3 + dx
                col_ref[slot, pl.ds(t * 16, 16), :] = c2_ref[
                    :, pl.ds(o + (dy - 1) * rl_out + (dx - 1), cb)]
        y = jnp.dot(w2, col_ref[slot, pl.ds(0, 144), :],
                    preferred_element_type=f32)
        y = jnp.maximum(y + sh2, 0.0) * keep
        c3_ref[:, pl.ds(o, cb)] = y.astype(bf16)

    w3 = w3_ref[...]
    sc3 = sc3_ref[...]
    sh3 = sh3_ref[...]
    for j in range(6):
        slot = j % 2
        o = 2 * rl_out + j * cb
        for dy in range(3):
            for dx in range(3):
                t = dy * 3 + dx
                col_ref[slot, pl.ds(t * 24, 24), :] = c3_ref[
                    :, pl.ds(o + (dy - 1) * rl_out + (dx - 1), cb)]
        y = jnp.dot(w3, col_ref[slot, pl.ds(0, 216), :],
                    preferred_element_type=f32)
        y = jnp.maximum(y * sc3 + sh3, 0.0)
        for c in range(4):
            prod = y * wfc_ref[c, :, pl.ds(j * cb, cb)]
            if j == 0:
                facc_ref[c] = prod
            else:
                facc_ref[c] = facc_ref[c] + prod

    sums = [jnp.sum(facc_ref[c], axis=(0, 1), keepdims=True)
            for c in range(4)]
    row = jnp.concatenate(sums, axis=1) + fcb_ref[...]
    row = jnp.concatenate([row, jnp.zeros((1, 124), f32)], axis=1)
    out_ref[0] = jnp.broadcast_to(row, (8, 128))


def kernel(x_nchw, w1, b1, bn1_gamma, bn1_beta, bn1_mean, bn1_var,
           w2, b2, w3, b3, bn3_gamma, bn3_beta, bn3_mean, bn3_var,
           fc_w_packed, fc_b):
    n = x_nchw.shape[0]
    f32 = jnp.float32
    bf16 = jnp.bfloat16

    h, w_img, rl_in = 192, 192, 256
    canvas = jnp.zeros((n, 8, h + 4, rl_in), bf16)
    canvas = canvas.at[:, :3, 2:h + 2, 1:w_img + 1].set(x_nchw.astype(bf16))
    canvas = canvas.reshape(n, 8, (h + 4) * rl_in)

    def fold_w(w_hwio, cout8, cin8):
        cin, cout = w_hwio.shape[2], w_hwio.shape[3]
        wm = jnp.zeros((cout8, 3, 3, cin8), f32)
        wm = wm.at[:cout, :, :, :cin].set(
            jnp.transpose(w_hwio, (3, 0, 1, 2)).astype(f32))
        return wm.reshape(cout8, 9 * cin8)

    w1m = fold_w(w1, 16, 8).astype(bf16)
    w2m = fold_w(w2, 24, 16).astype(bf16)
    w3m = fold_w(w3, 32, 24).astype(bf16)

    def colvec(v, cout8):
        return jnp.zeros((cout8, 1), f32).at[:v.shape[0], 0].set(
            v.astype(f32))

    s1 = bn1_gamma * jax.lax.rsqrt(bn1_var + EPS)
    t1 = (b1 - bn1_mean) * s1 + bn1_beta
    s3 = bn3_gamma * jax.lax.rsqrt(bn3_var + EPS)
    t3 = (b3 - bn3_mean) * s3 + bn3_beta
    sc1, sh1 = colvec(s1, 16), colvec(t1, 16)
    sh2 = colvec(b2, 24)
    sc3, sh3 = colvec(s3, 32), colvec(t3, 32)

    sel_np = np.zeros((256, 256), np.float32)
    for c in range(1, 97):
        sel_np[2 * c - 1, c] = 1.0
        sel_np[2 * c, 128 + c] = 1.0
    sel = jnp.asarray(sel_np).astype(bf16)

    wfc = fc_w_packed.reshape(4, 32, 100, 128)[:, :, 2:98, :]
    wfc = wfc.reshape(4, 32, 96 * 128).astype(f32)
    fcb = fc_b.reshape(1, 4).astype(f32)

    flat1 = (h + 4) * rl_in
    out = pl.pallas_call(
        _fused_kernel,
        out_shape=jax.ShapeDtypeStruct((n, 8, 128), f32),
        grid=(n,),
        in_specs=[
            pl.BlockSpec((1, 8, flat1), lambda b: (b, 0, 0)),
            pl.BlockSpec((16, 72), lambda b: (0, 0)),
            pl.BlockSpec((16, 1), lambda b: (0, 0)),
            pl.BlockSpec((16, 1), lambda b: (0, 0)),
            pl.BlockSpec((256, 256), lambda b: (0, 0)),
            pl.BlockSpec((24, 144), lambda b: (0, 0)),
            pl.BlockSpec((24, 1), lambda b: (0, 0)),
            pl.BlockSpec((32, 216), lambda b: (0, 0)),
            pl.BlockSpec((32, 1), lambda b: (0, 0)),
            pl.BlockSpec((32, 1), lambda b: (0, 0)),
            pl.BlockSpec((4, 32, 96 * 128), lambda b: (0, 0, 0)),
            pl.BlockSpec((1, 4), lambda b: (0, 0)),
        ],
        out_specs=pl.BlockSpec((1, 8, 128), lambda b: (b, 0, 0)),
        scratch_shapes=[
            pltpu.VMEM((2, 216, 2048), bf16),
            pltpu.VMEM((16, 100 * 128), bf16),
            pltpu.VMEM((24, 100 * 128), bf16),
            pltpu.VMEM((64, 256), bf16),
            pltpu.VMEM((4, 32, 2048), f32),
        ],
        compiler_params=pltpu.CompilerParams(
            dimension_semantics=("parallel",),
            vmem_limit_bytes=VMEM_LIMIT),
    )(canvas, w1m, sc1, sh1, sel, w2m, sh2, w3m, sc3, sh3, wfc, fcb)
    return out[:, 0, :4]


# speedup vs baseline: 1.0310x; 1.0310x over previous
import functools

import jax
import jax.numpy as jnp
import numpy as np
from jax.experimental import pallas as pl
from jax.experimental.pallas import tpu as pltpu

EPS = 1e-5
VMEM_LIMIT = 64 * 1024 * 1024


def _fused_kernel(xr_ref, w1_ref, sc1_ref, sh1_ref, sel_ref,
                  w2_ref, sh2_ref, w3_ref, sc3_ref, sh3_ref,
                  wfc_ref, fcb_ref, out_ref,
                  canvas_ref, col_ref, c2_ref, c3_ref, stack_ref, facc_ref):
    f32 = jnp.float32
    bf16 = jnp.bfloat16

    rl_in, rl_out = 256, 128
    cb1 = 2048
    canvas_ref[:, pl.ds(0, 2 * rl_in)] = jnp.zeros((8, 2 * rl_in), bf16)
    canvas_ref[:, pl.ds(194 * rl_in, 2 * rl_in)] = jnp.zeros((8, 2 * rl_in),
                                                             bf16)
    for g in range(24):
        canvas_ref[:, pl.ds((2 + 8 * g) * rl_in, cb1)] = xr_ref[
            0, :, pl.ds(g * cb1, cb1)]

    w1 = w1_ref[...]
    sc1 = sc1_ref[...]
    sh1 = sh1_ref[...]
    sel = sel_ref[...]

    c2_ref[:, pl.ds(0, 2 * rl_out)] = jnp.zeros((16, 2 * rl_out), bf16)
    c2_ref[:, pl.ds(98 * rl_out, 2 * rl_out)] = jnp.zeros((16, 2 * rl_out), bf16)

    for g in range(24):
        slot = g % 2
        o = 2 * rl_in + g * cb1
        for dy in range(3):
            for dx in range(3):
                t = dy * 3 + dx
                col_ref[slot, pl.ds(t * 8, 8), :] = canvas_ref[
                    :, pl.ds(o + (dy - 1) * rl_in + (dx - 1), cb1)]
        y = jnp.dot(w1, col_ref[slot, pl.ds(0, 72), :],
                    preferred_element_type=f32)
        y = jnp.maximum(y * sc1 + sh1, 0.0)
        for p in range(4):
            a = y[:, (2 * p) * rl_in:(2 * p + 1) * rl_in]
            b = y[:, (2 * p + 1) * rl_in:(2 * p + 2) * rl_in]
            stack_ref[pl.ds(p * 16, 16), :] = jnp.maximum(a, b).astype(bf16)
        res = jnp.dot(stack_ref[...], sel, preferred_element_type=f32)
        pooled = jnp.maximum(res[:, :rl_out], res[:, rl_out:])
        for p in range(4):
            yq = 2 + 4 * g + p
            c2_ref[:, pl.ds(yq * rl_out, rl_out)] = pooled[
                p * 16:(p + 1) * 16, :].astype(bf16)

    cb = 2048
    w2 = w2_ref[...]
    sh2 = sh2_ref[...]
    lane = jax.lax.broadcasted_iota(jnp.int32, (1, cb), 1) % rl_out
    keep = (lane <= 95).astype(f32)

    c3_ref[:, pl.ds(0, 2 * rl_out)] = jnp.zeros((24, 2 * rl_out), bf16)
    c3_ref[:, pl.ds(98 * rl_out, 2 * rl_out)] = jnp.zeros((24, 2 * rl_out), bf16)

    for j in range(6):
        slot = j % 2
        o = 2 * rl_out + j * cb
        for dy in range(3):
            for dx in range(3):
                t = dy * 3 + dx
                col_ref[slot, pl.ds(t * 16, 16), :] = c2_ref[
                    :, pl.ds(o + (dy - 1) * rl_out + (dx - 1), cb)]
        y = jnp.dot(w2, col_ref[slot, pl.ds(0, 144), :],
                    preferred_element_type=f32)
        y = jnp.maximum(y + sh2, 0.0) * keep
        c3_ref[:, pl.ds(o, cb)] = y.astype(bf16)

    w3 = w3_ref[...]
    sc3 = sc3_ref[...]
    sh3 = sh3_ref[...]
    for j in range(6):
        slot = j % 2
        o = 2 * rl_out + j * cb
        for dy in range(3):
            for dx in range(3):
                t = dy * 3 + dx
                col_ref[slot, pl.ds(t * 24, 24), :] = c3_ref[
                    :, pl.ds(o + (dy - 1) * rl_out + (dx - 1), cb)]
        y = jnp.dot(w3, col_ref[slot, pl.ds(0, 216), :],
                    preferred_element_type=f32)
        y = jnp.maximum(y * sc3 + sh3, 0.0)
        for c in range(4):
            prod = y * wfc_ref[c, :, pl.ds(j * cb, cb)]
            if j == 0:
                facc_ref[c] = prod
            else:
                facc_ref[c] = facc_ref[c] + prod

    sums = [jnp.sum(facc_ref[c], axis=(0, 1), keepdims=True)
            for c in range(4)]
    row = jnp.concatenate(sums, axis=1) + fcb_ref[...]
    row = jnp.concatenate([row, jnp.zeros((1, 124), f32)], axis=1)
    out_ref[0] = jnp.broadcast_to(row, (8, 128))


def kernel(x_nchw, w1, b1, bn1_gamma, bn1_beta, bn1_mean, bn1_var,
           w2, b2, w3, b3, bn3_gamma, bn3_beta, bn3_mean, bn3_var,
           fc_w_packed, fc_b):
    n = x_nchw.shape[0]
    f32 = jnp.float32
    bf16 = jnp.bfloat16

    h, w_img, rl_in = 192, 192, 256
    xr = jnp.pad(x_nchw.astype(bf16), ((0, 0), (0, 5), (0, 0), (0, 64)))
    xr = xr.reshape(n, 8, h * rl_in)

    def fold_w(w_hwio, cout8, cin8):
        cin, cout = w_hwio.shape[2], w_hwio.shape[3]
        wm = jnp.zeros((cout8, 3, 3, cin8), f32)
        wm = wm.at[:cout, :, :, :cin].set(
            jnp.transpose(w_hwio, (3, 0, 1, 2)).astype(f32))
        return wm.reshape(cout8, 9 * cin8)

    w1m = fold_w(w1, 16, 8).astype(bf16)
    w2m = fold_w(w2, 24, 16).astype(bf16)
    w3m = fold_w(w3, 32, 24).astype(bf16)

    def colvec(v, cout8):
        return jnp.zeros((cout8, 1), f32).at[:v.shape[0], 0].set(
            v.astype(f32))

    s1 = bn1_gamma * jax.lax.rsqrt(bn1_var + EPS)
    t1 = (b1 - bn1_mean) * s1 + bn1_beta
    s3 = bn3_gamma * jax.lax.rsqrt(bn3_var + EPS)
    t3 = (b3 - bn3_mean) * s3 + bn3_beta
    sc1, sh1 = colvec(s1, 16), colvec(t1, 16)
    sh2 = colvec(b2, 24)
    sc3, sh3 = colvec(s3, 32), colvec(t3, 32)

    sel_np = np.zeros((256, 256), np.float32)
    for c in range(96):
        sel_np[2 * c, c] = 1.0
        sel_np[2 * c + 1, 128 + c] = 1.0
    sel = jnp.asarray(sel_np).astype(bf16)

    wfc = jnp.zeros((4, 32, 96, 128), f32)
    wfc = wfc.at[:, :, :, :96].set(
        fc_w_packed.reshape(4, 32, 100, 128)[:, :, 2:98, 1:97].astype(f32))
    wfc = wfc.reshape(4, 32, 96 * 128)
    fcb = fc_b.reshape(1, 4).astype(f32)

    out = pl.pallas_call(
        _fused_kernel,
        out_shape=jax.ShapeDtypeStruct((n, 8, 128), f32),
        grid=(n,),
        in_specs=[
            pl.BlockSpec((1, 8, h * rl_in), lambda b: (b, 0, 0)),
            pl.BlockSpec((16, 72), lambda b: (0, 0)),
            pl.BlockSpec((16, 1), lambda b: (0, 0)),
            pl.BlockSpec((16, 1), lambda b: (0, 0)),
            pl.BlockSpec((256, 256), lambda b: (0, 0)),
            pl.BlockSpec((24, 144), lambda b: (0, 0)),
            pl.BlockSpec((24, 1), lambda b: (0, 0)),
            pl.BlockSpec((32, 216), lambda b: (0, 0)),
            pl.BlockSpec((32, 1), lambda b: (0, 0)),
            pl.BlockSpec((32, 1), lambda b: (0, 0)),
            pl.BlockSpec((4, 32, 96 * 128), lambda b: (0, 0, 0)),
            pl.BlockSpec((1, 4), lambda b: (0, 0)),
        ],
        out_specs=pl.BlockSpec((1, 8, 128), lambda b: (b, 0, 0)),
        scratch_shapes=[
            pltpu.VMEM((8, 196 * 256), bf16),
            pltpu.VMEM((2, 216, 2048), bf16),
            pltpu.VMEM((16, 100 * 128), bf16),
            pltpu.VMEM((24, 100 * 128), bf16),
            pltpu.VMEM((64, 256), bf16),
            pltpu.VMEM((4, 32, 2048), f32),
        ],
        compiler_params=pltpu.CompilerParams(
            dimension_semantics=("parallel",),
            vmem_limit_bytes=VMEM_LIMIT),
    )(xr, w1m, sc1, sh1, sel, w2m, sh2, w3m, sc3, sh3, wfc, fcb)
    return out[:, 0, :4]


# speedup vs baseline: 1.1165x; 1.0830x over previous
import functools

import jax
import jax.numpy as jnp
import numpy as np
from jax.experimental import pallas as pl
from jax.experimental.pallas import tpu as pltpu

EPS = 1e-5
VMEM_LIMIT = 64 * 1024 * 1024


def _fused_kernel(xr_ref, w1_ref, sc1_ref, sh1_ref, sel_ref,
                  w2_ref, sh2_ref, w3_ref, sc3_ref, sh3_ref,
                  wfc_ref, fcb_ref, out_ref,
                  canvas_ref, col_ref, c2_ref, c3_ref, stack_ref, facc_ref):
    f32 = jnp.float32
    bf16 = jnp.bfloat16

    rl_in, rl_out = 256, 128
    cb1 = 2048
    canvas_ref[...] = jnp.zeros(canvas_ref.shape, bf16)
    for k in range(96):
        s = xr_ref[0, :, pl.ds(384 * k, 384)].astype(bf16)
        canvas_ref[0:3, pl.ds((2 + 2 * k) * rl_in, 192)] = s[:, 0:192]
        canvas_ref[0:3, pl.ds((3 + 2 * k) * rl_in, 192)] = s[:, 192:384]

    w1 = w1_ref[...]
    sc1 = sc1_ref[...]
    sh1 = sh1_ref[...]
    sel = sel_ref[...]

    c2_ref[:, pl.ds(0, 2 * rl_out)] = jnp.zeros((16, 2 * rl_out), bf16)
    c2_ref[:, pl.ds(98 * rl_out, 2 * rl_out)] = jnp.zeros((16, 2 * rl_out), bf16)

    for g in range(24):
        slot = g % 2
        o = 2 * rl_in + g * cb1
        for dy in range(3):
            for dx in range(3):
                t = dy * 3 + dx
                col_ref[slot, pl.ds(t * 8, 8), :] = canvas_ref[
                    :, pl.ds(o + (dy - 1) * rl_in + (dx - 1), cb1)]
        y = jnp.dot(w1, col_ref[slot, pl.ds(0, 72), :],
                    preferred_element_type=f32)
        y = jnp.maximum(y * sc1 + sh1, 0.0)
        for p in range(4):
            a = y[:, (2 * p) * rl_in:(2 * p + 1) * rl_in]
            b = y[:, (2 * p + 1) * rl_in:(2 * p + 2) * rl_in]
            stack_ref[pl.ds(p * 16, 16), :] = jnp.maximum(a, b).astype(bf16)
        res = jnp.dot(stack_ref[...], sel, preferred_element_type=f32)
        pooled = jnp.maximum(res[:, :rl_out], res[:, rl_out:])
        for p in range(4):
            yq = 2 + 4 * g + p
            c2_ref[:, pl.ds(yq * rl_out, rl_out)] = pooled[
                p * 16:(p + 1) * 16, :].astype(bf16)

    cb = 2048
    w2 = w2_ref[...]
    sh2 = sh2_ref[...]
    lane = jax.lax.broadcasted_iota(jnp.int32, (1, cb), 1) % rl_out
    keep = (lane <= 95).astype(f32)

    c3_ref[:, pl.ds(0, 2 * rl_out)] = jnp.zeros((24, 2 * rl_out), bf16)
    c3_ref[:, pl.ds(98 * rl_out, 2 * rl_out)] = jnp.zeros((24, 2 * rl_out), bf16)

    for j in range(6):
        slot = j % 2
        o = 2 * rl_out + j * cb
        for dy in range(3):
            for dx in range(3):
                t = dy * 3 + dx
                col_ref[slot, pl.ds(t * 16, 16), :] = c2_ref[
                    :, pl.ds(o + (dy - 1) * rl_out + (dx - 1), cb)]
        y = jnp.dot(w2, col_ref[slot, pl.ds(0, 144), :],
                    preferred_element_type=f32)
        y = jnp.maximum(y + sh2, 0.0) * keep
        c3_ref[:, pl.ds(o, cb)] = y.astype(bf16)

    w3 = w3_ref[...]
    sc3 = sc3_ref[...]
    sh3 = sh3_ref[...]
    for j in range(6):
        slot = j % 2
        o = 2 * rl_out + j * cb
        for dy in range(3):
            for dx in range(3):
                t = dy * 3 + dx
                col_ref[slot, pl.ds(t * 24, 24), :] = c3_ref[
                    :, pl.ds(o + (dy - 1) * rl_out + (dx - 1), cb)]
        y = jnp.dot(w3, col_ref[slot, pl.ds(0, 216), :],
                    preferred_element_type=f32)
        y = jnp.maximum(y * sc3 + sh3, 0.0)
        for c in range(4):
            prod = y * wfc_ref[c, :, pl.ds(j * cb, cb)]
            if j == 0:
                facc_ref[c] = prod
            else:
                facc_ref[c] = facc_ref[c] + prod

    sums = [jnp.sum(facc_ref[c], axis=(0, 1), keepdims=True)
            for c in range(4)]
    row = jnp.concatenate(sums, axis=1) + fcb_ref[...]
    row = jnp.concatenate([row, jnp.zeros((1, 124), f32)], axis=1)
    out_ref[0] = jnp.broadcast_to(row, (8, 128))


def kernel(x_nchw, w1, b1, bn1_gamma, bn1_beta, bn1_mean, bn1_var,
           w2, b2, w3, b3, bn3_gamma, bn3_beta, bn3_mean, bn3_var,
           fc_w_packed, fc_b):
    n = x_nchw.shape[0]
    f32 = jnp.float32
    bf16 = jnp.bfloat16

    h, w_img, rl_in = 192, 192, 256
    xr = x_nchw.reshape(n, 3, h * w_img)

    def fold_w(w_hwio, cout8, cin8):
        cin, cout = w_hwio.shape[2], w_hwio.shape[3]
        wm = jnp.zeros((cout8, 3, 3, cin8), f32)
        wm = wm.at[:cout, :, :, :cin].set(
            jnp.transpose(w_hwio, (3, 0, 1, 2)).astype(f32))
        return wm.reshape(cout8, 9 * cin8)

    w1m = fold_w(w1, 16, 8).astype(bf16)
    w2m = fold_w(w2, 24, 16).astype(bf16)
    w3m = fold_w(w3, 32, 24).astype(bf16)

    def colvec(v, cout8):
        return jnp.zeros((cout8, 1), f32).at[:v.shape[0], 0].set(
            v.astype(f32))

    s1 = bn1_gamma * jax.lax.rsqrt(bn1_var + EPS)
    t1 = (b1 - bn1_mean) * s1 + bn1_beta
    s3 = bn3_gamma * jax.lax.rsqrt(bn3_var + EPS)
    t3 = (b3 - bn3_mean) * s3 + bn3_beta
    sc1, sh1 = colvec(s1, 16), colvec(t1, 16)
    sh2 = colvec(b2, 24)
    sc3, sh3 = colvec(s3, 32), colvec(t3, 32)

    sel_np = np.zeros((256, 256), np.float32)
    for c in range(96):
        sel_np[2 * c, c] = 1.0
        sel_np[2 * c + 1, 128 + c] = 1.0
    sel = jnp.asarray(sel_np).astype(bf16)

    wfc = jnp.zeros((4, 32, 96, 128), f32)
    wfc = wfc.at[:, :, :, :96].set(
        fc_w_packed.reshape(4, 32, 100, 128)[:, :, 2:98, 1:97].astype(f32))
    wfc = wfc.reshape(4, 32, 96 * 128)
    fcb = fc_b.reshape(1, 4).astype(f32)

    out = pl.pallas_call(
        _fused_kernel,
        out_shape=jax.ShapeDtypeStruct((n, 8, 128), f32),
        grid=(n,),
        in_specs=[
            pl.BlockSpec((1, 3, h * w_img), lambda b: (b, 0, 0)),
            pl.BlockSpec((16, 72), lambda b: (0, 0)),
            pl.BlockSpec((16, 1), lambda b: (0, 0)),
            pl.BlockSpec((16, 1), lambda b: (0, 0)),
            pl.BlockSpec((256, 256), lambda b: (0, 0)),
            pl.BlockSpec((24, 144), lambda b: (0, 0)),
            pl.BlockSpec((24, 1), lambda b: (0, 0)),
            pl.BlockSpec((32, 216), lambda b: (0, 0)),
            pl.BlockSpec((32, 1), lambda b: (0, 0)),
            pl.BlockSpec((32, 1), lambda b: (0, 0)),
            pl.BlockSpec((4, 32, 96 * 128), lambda b: (0, 0, 0)),
            pl.BlockSpec((1, 4), lambda b: (0, 0)),
        ],
        out_specs=pl.BlockSpec((1, 8, 128), lambda b: (b, 0, 0)),
        scratch_shapes=[
            pltpu.VMEM((8, 196 * 256), bf16),
            pltpu.VMEM((2, 216, 2048), bf16),
            pltpu.VMEM((16, 100 * 128), bf16),
            pltpu.VMEM((24, 100 * 128), bf16),
            pltpu.VMEM((64, 256), bf16),
            pltpu.VMEM((4, 32, 2048), f32),
        ],
        compiler_params=pltpu.CompilerParams(
            dimension_semantics=("parallel",),
            vmem_limit_bytes=VMEM_LIMIT),
    )(xr, w1m, sc1, sh1, sel, w2m, sh2, w3m, sc3, sh3, wfc, fcb)
    return out[:, 0, :4]


# speedup vs baseline: 1.1299x; 1.0120x over previous
import functools

import jax
import jax.numpy as jnp
import numpy as np
from jax.experimental import pallas as pl
from jax.experimental.pallas import tpu as pltpu

EPS = 1e-5
VMEM_LIMIT = 64 * 1024 * 1024


def _fused_kernel(xr_ref, w1_ref, sc1_ref, sh1_ref, sel_ref,
                  w2_ref, sh2_ref, w3_ref, sc3_ref, sh3_ref,
                  wfc_ref, fcb_ref, out_ref,
                  canvas_ref, col_ref, c2_ref, c3_ref, stack_ref, facc_ref):
    f32 = jnp.float32
    bf16 = jnp.bfloat16

    rl_in, rl_out = 256, 128
    cb1 = 2048
    canvas_ref[...] = jnp.zeros(canvas_ref.shape, bf16)
    for k in range(96):
        s = xr_ref[0, :, pl.ds(384 * k, 384)].astype(bf16)
        canvas_ref[0:3, pl.ds((2 + 2 * k) * rl_in, 192)] = s[:, 0:192]
        canvas_ref[0:3, pl.ds((3 + 2 * k) * rl_in, 192)] = s[:, 192:384]

    w1 = w1_ref[...]
    sc1 = sc1_ref[...]
    sh1 = sh1_ref[...]
    sel = sel_ref[...]

    c2_ref[:, pl.ds(0, 2 * rl_out)] = jnp.zeros((16, 2 * rl_out), bf16)
    c2_ref[:, pl.ds(98 * rl_out, 2 * rl_out)] = jnp.zeros((16, 2 * rl_out), bf16)

    for g in range(24):
        slot = g % 2
        o = 2 * rl_in + g * cb1
        for dx in range(3):
            v = canvas_ref[:, pl.ds(o - rl_in + dx - 1, cb1 + 2 * rl_in)]
            for dy in range(3):
                col_ref[slot, pl.ds((dy * 3 + dx) * 8, 8), :] = v[
                    :, dy * rl_in:dy * rl_in + cb1]
        y = jnp.dot(w1, col_ref[slot, pl.ds(0, 72), :],
                    preferred_element_type=f32)
        y = jnp.maximum(y * sc1 + sh1, 0.0)
        for p in range(4):
            a = y[:, (2 * p) * rl_in:(2 * p + 1) * rl_in]
            b = y[:, (2 * p + 1) * rl_in:(2 * p + 2) * rl_in]
            stack_ref[pl.ds(p * 16, 16), :] = jnp.maximum(a, b).astype(bf16)
        res = jnp.dot(stack_ref[...], sel, preferred_element_type=f32)
        pooled = jnp.maximum(res[:, :rl_out], res[:, rl_out:])
        for p in range(4):
            yq = 2 + 4 * g + p
            c2_ref[:, pl.ds(yq * rl_out, rl_out)] = pooled[
                p * 16:(p + 1) * 16, :].astype(bf16)

    cb = 2048
    w2 = w2_ref[...]
    sh2 = sh2_ref[...]
    lane = jax.lax.broadcasted_iota(jnp.int32, (1, cb), 1) % rl_out
    keep = (lane <= 95).astype(f32)

    c3_ref[:, pl.ds(0, 2 * rl_out)] = jnp.zeros((24, 2 * rl_out), bf16)
    c3_ref[:, pl.ds(98 * rl_out, 2 * rl_out)] = jnp.zeros((24, 2 * rl_out), bf16)

    for j in range(6):
        slot = j % 2
        o = 2 * rl_out + j * cb
        for dx in range(3):
            v = c2_ref[:, pl.ds(o - rl_out + dx - 1, cb + 2 * rl_out)]
            for dy in range(3):
                col_ref[slot, pl.ds((dy * 3 + dx) * 16, 16), :] = v[
                    :, dy * rl_out:dy * rl_out + cb]
        y = jnp.dot(w2, col_ref[slot, pl.ds(0, 144), :],
                    preferred_element_type=f32)
        y = jnp.maximum(y + sh2, 0.0) * keep
        c3_ref[:, pl.ds(o, cb)] = y.astype(bf16)

    w3 = w3_ref[...]
    sc3 = sc3_ref[...]
    sh3 = sh3_ref[...]
    for j in range(6):
        slot = j % 2
        o = 2 * rl_out + j * cb
        for dx in range(3):
            v = c3_ref[:, pl.ds(o - rl_out + dx - 1, cb + 2 * rl_out)]
            for dy in range(3):
                col_ref[slot, pl.ds((dy * 3 + dx) * 24, 24), :] = v[
                    :, dy * rl_out:dy * rl_out + cb]
        y = jnp.dot(w3, col_ref[slot, pl.ds(0, 216), :],
                    preferred_element_type=f32)
        y = jnp.maximum(y * sc3 + sh3, 0.0)
        for c in range(4):
            prod = y * wfc_ref[c, :, pl.ds(j * cb, cb)]
            if j == 0:
                facc_ref[c] = prod
            else:
                facc_ref[c] = facc_ref[c] + prod

    sums = [jnp.sum(facc_ref[c], axis=(0, 1), keepdims=True)
            for c in range(4)]
    row = jnp.concatenate(sums, axis=1) + fcb_ref[...]
    row = jnp.concatenate([row, jnp.zeros((1, 124), f32)], axis=1)
    out_ref[0] = jnp.broadcast_to(row, (8, 128))


def kernel(x_nchw, w1, b1, bn1_gamma, bn1_beta, bn1_mean, bn1_var,
           w2, b2, w3, b3, bn3_gamma, bn3_beta, bn3_mean, bn3_var,
           fc_w_packed, fc_b):
    n = x_nchw.shape[0]
    f32 = jnp.float32
    bf16 = jnp.bfloat16

    h, w_img, rl_in = 192, 192, 256
    xr = x_nchw.reshape(n, 3, h * w_img)

    def fold_w(w_hwio, cout8, cin8):
        cin, cout = w_hwio.shape[2], w_hwio.shape[3]
        wm = jnp.zeros((cout8, 3, 3, cin8), f32)
        wm = wm.at[:cout, :, :, :cin].set(
            jnp.transpose(w_hwio, (3, 0, 1, 2)).astype(f32))
        return wm.reshape(cout8, 9 * cin8)

    w1m = fold_w(w1, 16, 8).astype(bf16)
    w2m = fold_w(w2, 24, 16).astype(bf16)
    w3m = fold_w(w3, 32, 24).astype(bf16)

    def colvec(v, cout8):
        return jnp.zeros((cout8, 1), f32).at[:v.shape[0], 0].set(
            v.astype(f32))

    s1 = bn1_gamma * jax.lax.rsqrt(bn1_var + EPS)
    t1 = (b1 - bn1_mean) * s1 + bn1_beta
    s3 = bn3_gamma * jax.lax.rsqrt(bn3_var + EPS)
    t3 = (b3 - bn3_mean) * s3 + bn3_beta
    sc1, sh1 = colvec(s1, 16), colvec(t1, 16)
    sh2 = colvec(b2, 24)
    sc3, sh3 = colvec(s3, 32), colvec(t3, 32)

    sel_np = np.zeros((256, 256), np.float32)
    for c in range(96):
        sel_np[2 * c, c] = 1.0
        sel_np[2 * c + 1, 128 + c] = 1.0
    sel = jnp.asarray(sel_np).astype(bf16)

    wfc = jnp.zeros((4, 32, 96, 128), f32)
    wfc = wfc.at[:, :, :, :96].set(
        fc_w_packed.reshape(4, 32, 100, 128)[:, :, 2:98, 1:97].astype(f32))
    wfc = wfc.reshape(4, 32, 96 * 128)
    fcb = fc_b.reshape(1, 4).astype(f32)

    out = pl.pallas_call(
        _fused_kernel,
        out_shape=jax.ShapeDtypeStruct((n, 8, 128), f32),
        grid=(n,),
        in_specs=[
            pl.BlockSpec((1, 3, h * w_img), lambda b: (b, 0, 0)),
            pl.BlockSpec((16, 72), lambda b: (0, 0)),
            pl.BlockSpec((16, 1), lambda b: (0, 0)),
            pl.BlockSpec((16, 1), lambda b: (0, 0)),
            pl.BlockSpec((256, 256), lambda b: (0, 0)),
            pl.BlockSpec((24, 144), lambda b: (0, 0)),
            pl.BlockSpec((24, 1), lambda b: (0, 0)),
            pl.BlockSpec((32, 216), lambda b: (0, 0)),
            pl.BlockSpec((32, 1), lambda b: (0, 0)),
            pl.BlockSpec((32, 1), lambda b: (0, 0)),
            pl.BlockSpec((4, 32, 96 * 128), lambda b: (0, 0, 0)),
            pl.BlockSpec((1, 4), lambda b: (0, 0)),
        ],
        out_specs=pl.BlockSpec((1, 8, 128), lambda b: (b, 0, 0)),
        scratch_shapes=[
            pltpu.VMEM((8, 196 * 256), bf16),
            pltpu.VMEM((2, 216, 2048), bf16),
            pltpu.VMEM((16, 100 * 128), bf16),
            pltpu.VMEM((24, 100 * 128), bf16),
            pltpu.VMEM((64, 256), bf16),
            pltpu.VMEM((4, 32, 2048), f32),
        ],
        compiler_params=pltpu.CompilerParams(
            dimension_semantics=("parallel",),
            vmem_limit_bytes=VMEM_LIMIT),
    )(xr, w1m, sc1, sh1, sel, w2m, sh2, w3m, sc3, sh3, wfc, fcb)
    return out[:, 0, :4]


# speedup vs baseline: 1.2718x; 1.1255x over previous
import functools

import jax
import jax.numpy as jnp
import numpy as np
from jax.experimental import pallas as pl
from jax.experimental.pallas import tpu as pltpu

EPS = 1e-5
VMEM_LIMIT = 64 * 1024 * 1024


def _fused_kernel(xr_ref, w1_ref, sc1_ref, sh1_ref, sel_ref,
                  w2_ref, sh2_ref, w3_ref, sc3_ref, sh3_ref,
                  wfc_ref, fcb_ref, out_ref,
                  canvas_ref, col_ref, c2_ref, c3_ref, stack_ref, facc_ref):
    f32 = jnp.float32
    bf16 = jnp.bfloat16

    rl_in, rl_out = 256, 128
    cb1 = 2048
    canvas_ref[...] = jnp.zeros(canvas_ref.shape, bf16)
    for g in range(24):
        t8 = xr_ref[0, :, pl.ds(8 * g, 8), :].astype(bf16)
        for r in range(8):
            canvas_ref[0:3, pl.ds((2 + 8 * g + r) * rl_in, 192)] = t8[:, r, :]

    w1 = w1_ref[...]
    sc1 = sc1_ref[...]
    sh1 = sh1_ref[...]
    sel = sel_ref[...]

    c2_ref[:, pl.ds(0, 2 * rl_out)] = jnp.zeros((16, 2 * rl_out), bf16)
    c2_ref[:, pl.ds(98 * rl_out, 2 * rl_out)] = jnp.zeros((16, 2 * rl_out), bf16)

    for g in range(24):
        slot = g % 2
        o = 2 * rl_in + g * cb1
        for dx in range(3):
            v = canvas_ref[:, pl.ds(o - rl_in + dx - 1, cb1 + 2 * rl_in)]
            for dy in range(3):
                col_ref[slot, pl.ds((dy * 3 + dx) * 8, 8), :] = v[
                    :, dy * rl_in:dy * rl_in + cb1]
        y = jnp.dot(w1, col_ref[slot, pl.ds(0, 72), :],
                    preferred_element_type=f32)
        y = jnp.maximum(y * sc1 + sh1, 0.0)
        for p in range(4):
            a = y[:, (2 * p) * rl_in:(2 * p + 1) * rl_in]
            b = y[:, (2 * p + 1) * rl_in:(2 * p + 2) * rl_in]
            stack_ref[pl.ds(p * 16, 16), :] = jnp.maximum(a, b).astype(bf16)
        res = jnp.dot(stack_ref[...], sel, preferred_element_type=f32)
        pooled = jnp.maximum(res[:, :rl_out], res[:, rl_out:])
        for p in range(4):
            yq = 2 + 4 * g + p
            c2_ref[:, pl.ds(yq * rl_out, rl_out)] = pooled[
                p * 16:(p + 1) * 16, :].astype(bf16)

    cb = 2048
    w2 = w2_ref[...]
    sh2 = sh2_ref[...]
    lane = jax.lax.broadcasted_iota(jnp.int32, (1, cb), 1) % rl_out
    keep = (lane <= 95).astype(f32)

    c3_ref[:, pl.ds(0, 2 * rl_out)] = jnp.zeros((24, 2 * rl_out), bf16)
    c3_ref[:, pl.ds(98 * rl_out, 2 * rl_out)] = jnp.zeros((24, 2 * rl_out), bf16)

    for j in range(6):
        slot = j % 2
        o = 2 * rl_out + j * cb
        for dx in range(3):
            v = c2_ref[:, pl.ds(o - rl_out + dx - 1, cb + 2 * rl_out)]
            for dy in range(3):
                col_ref[slot, pl.ds((dy * 3 + dx) * 16, 16), :] = v[
                    :, dy * rl_out:dy * rl_out + cb]
        y = jnp.dot(w2, col_ref[slot, pl.ds(0, 144), :],
                    preferred_element_type=f32)
        y = jnp.maximum(y + sh2, 0.0) * keep
        c3_ref[:, pl.ds(o, cb)] = y.astype(bf16)

    w3 = w3_ref[...]
    sc3 = sc3_ref[...]
    sh3 = sh3_ref[...]
    for j in range(6):
        slot = j % 2
        o = 2 * rl_out + j * cb
        for dx in range(3):
            v = c3_ref[:, pl.ds(o - rl_out + dx - 1, cb + 2 * rl_out)]
            for dy in range(3):
                col_ref[slot, pl.ds((dy * 3 + dx) * 24, 24), :] = v[
                    :, dy * rl_out:dy * rl_out + cb]
        y = jnp.dot(w3, col_ref[slot, pl.ds(0, 216), :],
                    preferred_element_type=f32)
        y = jnp.maximum(y * sc3 + sh3, 0.0)
        for c in range(4):
            prod = y * wfc_ref[c, :, pl.ds(j * cb, cb)]
            if j == 0:
                facc_ref[c] = prod
            else:
                facc_ref[c] = facc_ref[c] + prod

    sums = [jnp.sum(facc_ref[c], axis=(0, 1), keepdims=True)
            for c in range(4)]
    row = jnp.concatenate(sums, axis=1) + fcb_ref[...]
    row = jnp.concatenate([row, jnp.zeros((1, 124), f32)], axis=1)
    out_ref[0] = jnp.broadcast_to(row, (8, 128))


def kernel(x_nchw, w1, b1, bn1_gamma, bn1_beta, bn1_mean, bn1_var,
           w2, b2, w3, b3, bn3_gamma, bn3_beta, bn3_mean, bn3_var,
           fc_w_packed, fc_b):
    n = x_nchw.shape[0]
    f32 = jnp.float32
    bf16 = jnp.bfloat16

    h, w_img, rl_in = 192, 192, 256

    def fold_w(w_hwio, cout8, cin8):
        cin, cout = w_hwio.shape[2], w_hwio.shape[3]
        wm = jnp.zeros((cout8, 3, 3, cin8), f32)
        wm = wm.at[:cout, :, :, :cin].set(
            jnp.transpose(w_hwio, (3, 0, 1, 2)).astype(f32))
        return wm.reshape(cout8, 9 * cin8)

    w1m = fold_w(w1, 16, 8).astype(bf16)
    w2m = fold_w(w2, 24, 16).astype(bf16)
    w3m = fold_w(w3, 32, 24).astype(bf16)

    def colvec(v, cout8):
        return jnp.zeros((cout8, 1), f32).at[:v.shape[0], 0].set(
            v.astype(f32))

    s1 = bn1_gamma * jax.lax.rsqrt(bn1_var + EPS)
    t1 = (b1 - bn1_mean) * s1 + bn1_beta
    s3 = bn3_gamma * jax.lax.rsqrt(bn3_var + EPS)
    t3 = (b3 - bn3_mean) * s3 + bn3_beta
    sc1, sh1 = colvec(s1, 16), colvec(t1, 16)
    sh2 = colvec(b2, 24)
    sc3, sh3 = colvec(s3, 32), colvec(t3, 32)

    sel_np = np.zeros((256, 256), np.float32)
    for c in range(96):
        sel_np[2 * c, c] = 1.0
        sel_np[2 * c + 1, 128 + c] = 1.0
    sel = jnp.asarray(sel_np).astype(bf16)

    wfc = jnp.zeros((4, 32, 96, 128), f32)
    wfc = wfc.at[:, :, :, :96].set(
        fc_w_packed.reshape(4, 32, 100, 128)[:, :, 2:98, 1:97].astype(f32))
    wfc = wfc.reshape(4, 32, 96 * 128)
    fcb = fc_b.reshape(1, 4).astype(f32)

    out = pl.pallas_call(
        _fused_kernel,
        out_shape=jax.ShapeDtypeStruct((n, 8, 128), f32),
        grid=(n,),
        in_specs=[
            pl.BlockSpec((1, 3, h, w_img), lambda b: (b, 0, 0, 0)),
            pl.BlockSpec((16, 72), lambda b: (0, 0)),
            pl.BlockSpec((16, 1), lambda b: (0, 0)),
            pl.BlockSpec((16, 1), lambda b: (0, 0)),
            pl.BlockSpec((256, 256), lambda b: (0, 0)),
            pl.BlockSpec((24, 144), lambda b: (0, 0)),
            pl.BlockSpec((24, 1), lambda b: (0, 0)),
            pl.BlockSpec((32, 216), lambda b: (0, 0)),
            pl.BlockSpec((32, 1), lambda b: (0, 0)),
            pl.BlockSpec((32, 1), lambda b: (0, 0)),
            pl.BlockSpec((4, 32, 96 * 128), lambda b: (0, 0, 0)),
            pl.BlockSpec((1, 4), lambda b: (0, 0)),
        ],
        out_specs=pl.BlockSpec((1, 8, 128), lambda b: (b, 0, 0)),
        scratch_shapes=[
            pltpu.VMEM((8, 196 * 256), bf16),
            pltpu.VMEM((2, 216, 2048), bf16),
            pltpu.VMEM((16, 100 * 128), bf16),
            pltpu.VMEM((24, 100 * 128), bf16),
            pltpu.VMEM((64, 256), bf16),
            pltpu.VMEM((4, 32, 2048), f32),
        ],
        compiler_params=pltpu.CompilerParams(
            dimension_semantics=("parallel",),
            vmem_limit_bytes=VMEM_LIMIT),
    )(x_nchw, w1m, sc1, sh1, sel, w2m, sh2, w3m, sc3, sh3, wfc, fcb)
    return out[:, 0, :4]


# speedup vs baseline: 1.2767x; 1.0039x over previous
import jax
import jax.numpy as jnp
import numpy as np
from jax.experimental import pallas as pl
from jax.experimental.pallas import tpu as pltpu

EPS = 1e-5
VMEM_LIMIT = 64 * 1024 * 1024


def _fused_kernel(xr_ref, w1_ref, sc1_ref, sh1_ref, sel_ref,
                  w2_ref, sh2_ref, w3_ref, sc3_ref, sh3_ref,
                  wfc_ref, fcb_ref, out_ref,
                  canvas_ref, col_ref, c2_ref, c3_ref, stack_ref):
    f32 = jnp.float32
    bf16 = jnp.bfloat16

    rl_in, rl_out = 256, 128
    cb1 = 2048
    canvas_ref[...] = jnp.zeros(canvas_ref.shape, bf16)
    for g in range(24):
        t8 = xr_ref[0, :, pl.ds(8 * g, 8), :].astype(bf16)
        for r in range(8):
            canvas_ref[0:3, pl.ds((2 + 8 * g + r) * rl_in, 192)] = t8[:, r, :]

    w1 = w1_ref[...]
    sc1 = sc1_ref[...]
    sh1 = sh1_ref[...]
    sel = sel_ref[...]

    c2_ref[:, pl.ds(0, 2 * rl_out)] = jnp.zeros((16, 2 * rl_out), bf16)
    c2_ref[:, pl.ds(98 * rl_out, 2 * rl_out)] = jnp.zeros((16, 2 * rl_out), bf16)

    for g in range(24):
        slot = g % 2
        o = 2 * rl_in + g * cb1
        for dx in range(3):
            v = canvas_ref[:, pl.ds(o - rl_in + dx - 1, cb1 + 2 * rl_in)]
            for dy in range(3):
                col_ref[slot, pl.ds((dy * 3 + dx) * 8, 8), :] = v[
                    :, dy * rl_in:dy * rl_in + cb1]
        y = jnp.dot(w1, col_ref[slot, pl.ds(0, 72), :],
                    preferred_element_type=f32)
        y = jnp.maximum(y * sc1 + sh1, 0.0)
        for p in range(4):
            a = y[:, (2 * p) * rl_in:(2 * p + 1) * rl_in]
            b = y[:, (2 * p + 1) * rl_in:(2 * p + 2) * rl_in]
            stack_ref[pl.ds(p * 16, 16), :] = jnp.maximum(a, b).astype(bf16)
        res = jnp.dot(stack_ref[...], sel, preferred_element_type=f32)
        pooled = jnp.maximum(res[:, :rl_out], res[:, rl_out:])
        for p in range(4):
            yq = 2 + 4 * g + p
            c2_ref[:, pl.ds(yq * rl_out, rl_out)] = pooled[
                p * 16:(p + 1) * 16, :].astype(bf16)

    cb = 2048
    w2 = w2_ref[...]
    sh2 = sh2_ref[...]
    lane = jax.lax.broadcasted_iota(jnp.int32, (1, cb), 1) % rl_out
    keep = (lane <= 95).astype(f32)

    c3_ref[:, pl.ds(0, 2 * rl_out)] = jnp.zeros((24, 2 * rl_out), bf16)
    c3_ref[:, pl.ds(98 * rl_out, 2 * rl_out)] = jnp.zeros((24, 2 * rl_out), bf16)

    for j in range(6):
        slot = j % 2
        o = 2 * rl_out + j * cb
        for dx in range(3):
            v = c2_ref[:, pl.ds(o - rl_out + dx - 1, cb + 2 * rl_out)]
            for dy in range(3):
                col_ref[slot, pl.ds((dy * 3 + dx) * 16, 16), :] = v[
                    :, dy * rl_out:dy * rl_out + cb]
        y = jnp.dot(w2, col_ref[slot, pl.ds(0, 144), :],
                    preferred_element_type=f32)
        y = jnp.maximum(y + sh2, 0.0) * keep
        c3_ref[:, pl.ds(o, cb)] = y.astype(bf16)

    w3 = w3_ref[...]
    sc3 = sc3_ref[...]
    sh3 = sh3_ref[...]
    accs = [jnp.zeros((8, 128), f32) for _ in range(4)]
    for j in range(6):
        slot = j % 2
        o = 2 * rl_out + j * cb
        for dx in range(3):
            v = c3_ref[:, pl.ds(o - rl_out + dx - 1, cb + 2 * rl_out)]
            for dy in range(3):
                col_ref[slot, pl.ds((dy * 3 + dx) * 24, 24), :] = v[
                    :, dy * rl_out:dy * rl_out + cb]
        y = jnp.dot(w3, col_ref[slot, pl.ds(0, 216), :],
                    preferred_element_type=f32)
        y = jnp.maximum(y * sc3 + sh3, 0.0)
        for c in range(4):
            r = y * wfc_ref[c, :, pl.ds(j * cb, cb)]
            r = r[:, :1024] + r[:, 1024:]
            r = r[:, :512] + r[:, 512:]
            r = r[:, :256] + r[:, 256:]
            r = r[:, :128] + r[:, 128:]
            r = (r[0:8] + r[8:16]) + (r[16:24] + r[24:32])
            accs[c] = accs[c] + r

    sums = [jnp.sum(a, axis=(0, 1), keepdims=True) for a in accs]
    row = jnp.concatenate(sums, axis=1) + fcb_ref[...]
    row = jnp.concatenate([row, jnp.zeros((1, 124), f32)], axis=1)
    out_ref[0] = jnp.broadcast_to(row, (8, 128))


def kernel(x_nchw, w1, b1, bn1_gamma, bn1_beta, bn1_mean, bn1_var,
           w2, b2, w3, b3, bn3_gamma, bn3_beta, bn3_mean, bn3_var,
           fc_w_packed, fc_b):
    n = x_nchw.shape[0]
    f32 = jnp.float32
    bf16 = jnp.bfloat16

    h, w_img = 192, 192

    def fold_w(w_hwio, cout8, cin8):
        cin, cout = w_hwio.shape[2], w_hwio.shape[3]
        wm = jnp.zeros((cout8, 3, 3, cin8), f32)
        wm = wm.at[:cout, :, :, :cin].set(
            jnp.transpose(w_hwio, (3, 0, 1, 2)).astype(f32))
        return wm.reshape(cout8, 9 * cin8)

    w1m = fold_w(w1, 16, 8).astype(bf16)
    w2m = fold_w(w2, 24, 16).astype(bf16)
    w3m = fold_w(w3, 32, 24).astype(bf16)

    def colvec(v, cout8):
        return jnp.zeros((cout8, 1), f32).at[:v.shape[0], 0].set(
            v.astype(f32))

    s1 = bn1_gamma * jax.lax.rsqrt(bn1_var + EPS)
    t1 = (b1 - bn1_mean) * s1 + bn1_beta
    s3 = bn3_gamma * jax.lax.rsqrt(bn3_var + EPS)
    t3 = (b3 - bn3_mean) * s3 + bn3_beta
    sc1, sh1 = colvec(s1, 16), colvec(t1, 16)
    sh2 = colvec(b2, 24)
    sc3, sh3 = colvec(s3, 32), colvec(t3, 32)

    sel_np = np.zeros((256, 256), np.float32)
    for c in range(96):
        sel_np[2 * c, c] = 1.0
        sel_np[2 * c + 1, 128 + c] = 1.0
    sel = jnp.asarray(sel_np).astype(bf16)

    wfc = jnp.zeros((4, 32, 96, 128), f32)
    wfc = wfc.at[:, :, :, :96].set(
        fc_w_packed.reshape(4, 32, 100, 128)[:, :, 2:98, 1:97].astype(f32))
    wfc = wfc.reshape(4, 32, 96 * 128)
    fcb = fc_b.reshape(1, 4).astype(f32)

    out = pl.pallas_call(
        _fused_kernel,
        out_shape=jax.ShapeDtypeStruct((n, 8, 128), f32),
        grid=(n,),
        in_specs=[
            pl.BlockSpec((1, 3, h, w_img), lambda b: (b, 0, 0, 0)),
            pl.BlockSpec((16, 72), lambda b: (0, 0)),
            pl.BlockSpec((16, 1), lambda b: (0, 0)),
            pl.BlockSpec((16, 1), lambda b: (0, 0)),
            pl.BlockSpec((256, 256), lambda b: (0, 0)),
            pl.BlockSpec((24, 144), lambda b: (0, 0)),
            pl.BlockSpec((24, 1), lambda b: (0, 0)),
            pl.BlockSpec((32, 216), lambda b: (0, 0)),
            pl.BlockSpec((32, 1), lambda b: (0, 0)),
            pl.BlockSpec((32, 1), lambda b: (0, 0)),
            pl.BlockSpec((4, 32, 96 * 128), lambda b: (0, 0, 0)),
            pl.BlockSpec((1, 4), lambda b: (0, 0)),
        ],
        out_specs=pl.BlockSpec((1, 8, 128), lambda b: (b, 0, 0)),
        scratch_shapes=[
            pltpu.VMEM((8, 196 * 256), bf16),
            pltpu.VMEM((2, 216, 2048), bf16),
            pltpu.VMEM((16, 100 * 128), bf16),
            pltpu.VMEM((24, 100 * 128), bf16),
            pltpu.VMEM((64, 256), bf16),
        ],
        compiler_params=pltpu.CompilerParams(
            dimension_semantics=("parallel",),
            vmem_limit_bytes=VMEM_LIMIT),
    )(x_nchw, w1m, sc1, sh1, sel, w2m, sh2, w3m, sc3, sh3, wfc, fcb)
    return out[:, 0, :4]


# speedup vs baseline: 1.5476x; 1.2121x over previous
import jax
import jax.numpy as jnp
import numpy as np
from jax.experimental import pallas as pl
from jax.experimental.pallas import tpu as pltpu

EPS = 1e-5
VMEM_LIMIT = 64 * 1024 * 1024


def _fused_kernel(xr_ref, w1_ref, sc1_ref, sh1_ref, sel_ref,
                  w2_ref, sh2_ref, w3_ref, sc3_ref, sh3_ref,
                  wfc_ref, fcb_ref, out_ref,
                  canvas_ref, col_ref, c2_ref, c3_ref, stack_ref):
    f32 = jnp.float32
    bf16 = jnp.bfloat16

    rl_in, rl_out = 256, 128
    cb1 = 4096
    canvas_ref[...] = jnp.zeros(canvas_ref.shape, bf16)
    for g in range(24):
        t8 = xr_ref[0, :, pl.ds(8 * g, 8), :].astype(bf16)
        for r in range(8):
            canvas_ref[0:3, pl.ds((2 + 8 * g + r) * rl_in, 192)] = t8[:, r, :]

    w1 = w1_ref[...]
    sc1 = sc1_ref[...]
    sh1 = sh1_ref[...]
    sel = sel_ref[...]

    c2_ref[:, pl.ds(0, 2 * rl_out)] = jnp.zeros((16, 2 * rl_out), bf16)
    c2_ref[:, pl.ds(98 * rl_out, 2 * rl_out)] = jnp.zeros((16, 2 * rl_out), bf16)

    for g in range(12):
        slot = g % 2
        o = 2 * rl_in + g * cb1
        for dx in range(3):
            v = canvas_ref[:, pl.ds(o - rl_in + dx - 1, cb1 + 2 * rl_in)]
            for dy in range(3):
                col_ref[slot, pl.ds((dy * 3 + dx) * 8, 8), :] = v[
                    :, dy * rl_in:dy * rl_in + cb1]
        y = jnp.dot(w1, col_ref[slot, pl.ds(0, 72), :],
                    preferred_element_type=f32)
        y = jnp.maximum(y * sc1 + sh1, 0.0)
        for p in range(8):
            a = y[:, (2 * p) * rl_in:(2 * p + 1) * rl_in]
            b = y[:, (2 * p + 1) * rl_in:(2 * p + 2) * rl_in]
            stack_ref[pl.ds(p * 16, 16), :] = jnp.maximum(a, b).astype(bf16)
        res = jnp.dot(stack_ref[...], sel, preferred_element_type=f32)
        pooled = jnp.maximum(res[:, :rl_out], res[:, rl_out:])
        for p in range(8):
            yq = 2 + 8 * g + p
            c2_ref[:, pl.ds(yq * rl_out, rl_out)] = pooled[
                p * 16:(p + 1) * 16, :].astype(bf16)

    cb = 4096
    w2 = w2_ref[...]
    sh2 = sh2_ref[...]
    lane = jax.lax.broadcasted_iota(jnp.int32, (1, cb), 1) % rl_out
    keep = (lane <= 95).astype(f32)

    c3_ref[:, pl.ds(0, 2 * rl_out)] = jnp.zeros((24, 2 * rl_out), bf16)
    c3_ref[:, pl.ds(98 * rl_out, 2 * rl_out)] = jnp.zeros((24, 2 * rl_out), bf16)

    for j in range(3):
        slot = j % 2
        o = 2 * rl_out + j * cb
        for dx in range(3):
            v = c2_ref[:, pl.ds(o - rl_out + dx - 1, cb + 2 * rl_out)]
            for dy in range(3):
                col_ref[slot, pl.ds((dy * 3 + dx) * 16, 16), :] = v[
                    :, dy * rl_out:dy * rl_out + cb]
        y = jnp.dot(w2, col_ref[slot, pl.ds(0, 144), :],
                    preferred_element_type=f32)
        y = jnp.maximum(y + sh2, 0.0) * keep
        c3_ref[:, pl.ds(o, cb)] = y.astype(bf16)

    w3 = w3_ref[...]
    sc3 = sc3_ref[...]
    sh3 = sh3_ref[...]
    accs = [jnp.zeros((8, 128), f32) for _ in range(4)]
    for j in range(3):
        slot = j % 2
        o = 2 * rl_out + j * cb
        for dx in range(3):
            v = c3_ref[:, pl.ds(o - rl_out + dx - 1, cb + 2 * rl_out)]
            for dy in range(3):
                col_ref[slot, pl.ds((dy * 3 + dx) * 24, 24), :] = v[
                    :, dy * rl_out:dy * rl_out + cb]
        y = jnp.dot(w3, col_ref[slot, pl.ds(0, 216), :],
                    preferred_element_type=f32)
        y = jnp.maximum(y * sc3 + sh3, 0.0)
        for c in range(4):
            r = y * wfc_ref[c, :, pl.ds(j * cb, cb)]
            r = r[:, :2048] + r[:, 2048:]
            r = r[:, :1024] + r[:, 1024:]
            r = r[:, :512] + r[:, 512:]
            r = r[:, :256] + r[:, 256:]
            r = r[:, :128] + r[:, 128:]
            r = (r[0:8] + r[8:16]) + (r[16:24] + r[24:32])
            accs[c] = accs[c] + r

    sums = [jnp.sum(a, axis=(0, 1), keepdims=True) for a in accs]
    row = jnp.concatenate(sums, axis=1) + fcb_ref[...]
    row = jnp.concatenate([row, jnp.zeros((1, 124), f32)], axis=1)
    out_ref[0] = jnp.broadcast_to(row, (8, 128))


def kernel(x_nchw, w1, b1, bn1_gamma, bn1_beta, bn1_mean, bn1_var,
           w2, b2, w3, b3, bn3_gamma, bn3_beta, bn3_mean, bn3_var,
           fc_w_packed, fc_b):
    n = x_nchw.shape[0]
    f32 = jnp.float32
    bf16 = jnp.bfloat16

    h, w_img = 192, 192

    def fold_w(w_hwio, cout8, cin8):
        cin, cout = w_hwio.shape[2], w_hwio.shape[3]
        wm = jnp.zeros((cout8, 3, 3, cin8), f32)
        wm = wm.at[:cout, :, :, :cin].set(
            jnp.transpose(w_hwio, (3, 0, 1, 2)).astype(f32))
        return wm.reshape(cout8, 9 * cin8)

    w1m = fold_w(w1, 16, 8).astype(bf16)
    w2m = fold_w(w2, 24, 16).astype(bf16)
    w3m = fold_w(w3, 32, 24).astype(bf16)

    def colvec(v, cout8):
        return jnp.zeros((cout8, 1), f32).at[:v.shape[0], 0].set(
            v.astype(f32))

    s1 = bn1_gamma * jax.lax.rsqrt(bn1_var + EPS)
    t1 = (b1 - bn1_mean) * s1 + bn1_beta
    s3 = bn3_gamma * jax.lax.rsqrt(bn3_var + EPS)
    t3 = (b3 - bn3_mean) * s3 + bn3_beta
    sc1, sh1 = colvec(s1, 16), colvec(t1, 16)
    sh2 = colvec(b2, 24)
    sc3, sh3 = colvec(s3, 32), colvec(t3, 32)

    sel_np = np.zeros((256, 256), np.float32)
    for c in range(96):
        sel_np[2 * c, c] = 1.0
        sel_np[2 * c + 1, 128 + c] = 1.0
    sel = jnp.asarray(sel_np).astype(bf16)

    wfc = jnp.zeros((4, 32, 96, 128), f32)
    wfc = wfc.at[:, :, :, :96].set(
        fc_w_packed.reshape(4, 32, 100, 128)[:, :, 2:98, 1:97].astype(f32))
    wfc = wfc.reshape(4, 32, 96 * 128)
    fcb = fc_b.reshape(1, 4).astype(f32)

    out = pl.pallas_call(
        _fused_kernel,
        out_shape=jax.ShapeDtypeStruct((n, 8, 128), f32),
        grid=(n,),
        in_specs=[
            pl.BlockSpec((1, 3, h, w_img), lambda b: (b, 0, 0, 0)),
            pl.BlockSpec((16, 72), lambda b: (0, 0)),
            pl.BlockSpec((16, 1), lambda b: (0, 0)),
            pl.BlockSpec((16, 1), lambda b: (0, 0)),
            pl.BlockSpec((256, 256), lambda b: (0, 0)),
            pl.BlockSpec((24, 144), lambda b: (0, 0)),
            pl.BlockSpec((24, 1), lambda b: (0, 0)),
            pl.BlockSpec((32, 216), lambda b: (0, 0)),
            pl.BlockSpec((32, 1), lambda b: (0, 0)),
            pl.BlockSpec((32, 1), lambda b: (0, 0)),
            pl.BlockSpec((4, 32, 96 * 128), lambda b: (0, 0, 0)),
            pl.BlockSpec((1, 4), lambda b: (0, 0)),
        ],
        out_specs=pl.BlockSpec((1, 8, 128), lambda b: (b, 0, 0)),
        scratch_shapes=[
            pltpu.VMEM((8, 196 * 256), bf16),
            pltpu.VMEM((2, 216, 4096), bf16),
            pltpu.VMEM((16, 100 * 128), bf16),
            pltpu.VMEM((24, 100 * 128), bf16),
            pltpu.VMEM((128, 256), bf16),
        ],
        compiler_params=pltpu.CompilerParams(
            dimension_semantics=("parallel",),
            vmem_limit_bytes=VMEM_LIMIT),
    )(x_nchw, w1m, sc1, sh1, sel, w2m, sh2, w3m, sc3, sh3, wfc, fcb)
    return out[:, 0, :4]
